# baseline (speedup 1.0000x reference)
import functools

import jax
import jax.numpy as jnp
from jax import lax
from jax.experimental import pallas as pl
from jax.experimental.pallas import tpu as pltpu
from jax.experimental.pallas import tpu_sc as plsc

E = 8
K = 2
T = 2048
D = 1024
H = 2048
BC = 128
NB = T * K // BC + E
P = NB * BC
NC = 2
NS = 16
NW = NC * NS
RPW = T // NW
HR = RPW // 2
CDT = jnp.bfloat16


def _silu(h):
    return h * (1.0 / (1.0 + jnp.exp(-h)))


def _pack(xbf):
    n, d = xbf.shape
    lo = lax.bitcast_convert_type(xbf[:, :d // 2],
                                  jnp.int16).astype(jnp.int32) & 0xFFFF
    hi = lax.bitcast_convert_type(xbf[:, d // 2:],
                                  jnp.int16).astype(jnp.int32) << 16
    return lo | hi


def _unpack(xi):
    lo = lax.bitcast_convert_type((xi & 0xFFFF).astype(jnp.int16),
                                  jnp.bfloat16)
    hi = lax.bitcast_convert_type(
        lax.shift_right_logical(xi, 16).astype(jnp.int16), jnp.bfloat16)
    return jnp.concatenate([lo, hi], axis=1)


def _route_body(x_ref, gw_ref, gb_ref, bias_ref,
                s1_ref, s2_ref, g1_ref, g2_ref, be_ref, xh_ref):
    x = x_ref[...]
    xbf = x.astype(jnp.bfloat16)
    xh_ref[...] = _pack(xbf)
    scores = lax.dot_general(
        xbf, gw_ref[...].astype(jnp.bfloat16),
        (((1,), (1,)), ((), ())), preferred_element_type=jnp.float32)
    scores = scores + gb_ref[...] + bias_ref[...]
    lane = lax.broadcasted_iota(jnp.int32, (T, E), 1)
    m1 = jnp.max(scores, axis=1, keepdims=True)
    i1 = jnp.min(jnp.where(scores == m1, lane, E), axis=1, keepdims=True)
    sel1 = lane == i1
    scores2 = jnp.where(sel1, -jnp.inf, scores)
    m2 = jnp.max(scores2, axis=1, keepdims=True)
    i2 = jnp.min(jnp.where(scores2 == m2, lane, E), axis=1, keepdims=True)
    sel2 = lane == i2
    tt = jnp.exp(m2 - m1)
    g1_ref[...] = 1.0 / (1.0 + tt)
    g2_ref[...] = tt / (1.0 + tt)

    mask = jnp.where(sel1 | sel2, 1.0, 0.0)
    TB = 512
    r_i = lax.broadcasted_iota(jnp.int32, (TB, TB), 0)
    c_i = lax.broadcasted_iota(jnp.int32, (TB, TB), 1)
    ltri = jnp.where(c_i < r_i, 1.0, 0.0).astype(jnp.float32)
    carry = jnp.zeros((1, E), jnp.float32)
    ranks = []
    for j in range(T // TB):
        mb = mask[j * TB:(j + 1) * TB, :]
        rb = lax.dot_general(ltri, mb, (((1,), (0,)), ((), ())),
                             preferred_element_type=jnp.float32) + carry
        ranks.append(rb)
        carry = carry + jnp.sum(mb, axis=0, keepdims=True)
    rank = jnp.concatenate(ranks, axis=0)
    counts = carry
    padded = jnp.floor((counts + (BC - 1)) * (1.0 / BC)) * BC
    e_i = lax.broadcasted_iota(jnp.int32, (E, E), 0)
    e_j = lax.broadcasted_iota(jnp.int32, (E, E), 1)
    utri = jnp.where(e_i < e_j, 1.0, 0.0).astype(jnp.float32)
    off = lax.dot_general(padded, utri, (((1,), (0,)), ((), ())),
                          preferred_element_type=jnp.float32)
    bnd = (off + padded) * (1.0 / BC)
    b_i = lax.broadcasted_iota(jnp.int32, (NB, E), 0).astype(jnp.float32)
    be = jnp.sum(jnp.where(b_i >= bnd, 1.0, 0.0), axis=1, keepdims=True)
    be_ref[...] = jnp.minimum(be, E - 1).astype(jnp.int32)

    rank_o = rank + off
    s1 = jnp.sum(jnp.where(sel1, rank_o, 0.0), axis=1, keepdims=True)
    s2 = jnp.sum(jnp.where(sel2, rank_o, 0.0), axis=1, keepdims=True)
    s1_ref[...] = s1.astype(jnp.int32)
    s2_ref[...] = s2.astype(jnp.int32)


def _route(flat, gate_w, gate_b, bias):
    return pl.pallas_call(
        _route_body,
        out_shape=[
            jax.ShapeDtypeStruct((T, 1), jnp.int32),
            jax.ShapeDtypeStruct((T, 1), jnp.int32),
            jax.ShapeDtypeStruct((T, 1), jnp.float32),
            jax.ShapeDtypeStruct((T, 1), jnp.float32),
            jax.ShapeDtypeStruct((NB, 1), jnp.int32),
            jax.ShapeDtypeStruct((T, D // 2), jnp.int32),
        ],
    )(flat, gate_w, gate_b.reshape(1, E), bias.reshape(1, E))


def _dispatch_body(x_hbm, s1_hbm, s2_hbm, xg_hbm, xbuf, i1, i2, sem):
    wid = lax.axis_index("s") * NC + lax.axis_index("c")
    base = wid * RPW
    pltpu.sync_copy(x_hbm.at[pl.ds(base, RPW)], xbuf)
    pltpu.sync_copy(s1_hbm.at[pl.ds(base, RPW)], i1)
    pltpu.sync_copy(s2_hbm.at[pl.ds(base, RPW)], i2)
    pltpu.async_copy(xbuf, xg_hbm.at[i1], sem).wait()
    pltpu.async_copy(xbuf, xg_hbm.at[i2], sem).wait()


def _dispatch(flat, s1, s2):
    mesh = plsc.VectorSubcoreMesh(core_axis_name="c", subcore_axis_name="s")
    fn = functools.partial(
        pl.kernel, mesh=mesh,
        out_type=jax.ShapeDtypeStruct((P, D // 2), jnp.int32),
        scratch_types=[
            pltpu.VMEM((RPW, D // 2), jnp.int32),
            pltpu.VMEM((RPW,), jnp.int32),
            pltpu.VMEM((RPW,), jnp.int32),
            pltpu.SemaphoreType.DMA,
        ],
    )(_dispatch_body)
    return fn(flat, s1, s2)


def _ffn_body(be_ref, xg_ref, w1_ref, b1_ref, w2_ref, b2_ref, yg_ref):
    xb = _unpack(xg_ref[...])
    h = lax.dot_general(xb, w1_ref[0].astype(CDT),
                        (((1,), (0,)), ((), ())),
                        preferred_element_type=jnp.float32)
    h = _silu(h + b1_ref[0])
    y = lax.dot_general(h.astype(CDT), w2_ref[0].astype(CDT),
                        (((1,), (0,)), ((), ())),
                        preferred_element_type=jnp.float32)
    yb = (y + b2_ref[0]).astype(CDT)
    yg_ref[...] = _pack(yb)


def _grouped_ffn(be, xg, w1, b1, w2, b2):
    grid_spec = pltpu.PrefetchScalarGridSpec(
        num_scalar_prefetch=1,
        grid=(NB,),
        in_specs=[
            pl.BlockSpec((BC, D // 2), lambda b, be: (b, 0)),
            pl.BlockSpec((1, D, H), lambda b, be: (be[b], 0, 0)),
            pl.BlockSpec((1, 1, H), lambda b, be: (be[b], 0, 0)),
            pl.BlockSpec((1, H, D), lambda b, be: (be[b], 0, 0)),
            pl.BlockSpec((1, 1, D), lambda b, be: (be[b], 0, 0)),
        ],
        out_specs=pl.BlockSpec((BC, D // 2), lambda b, be: (b, 0)),
    )
    return pl.pallas_call(
        _ffn_body,
        grid_spec=grid_spec,
        out_shape=jax.ShapeDtypeStruct((P, D // 2), jnp.int32),
    )(be, xg, w1, b1.reshape(E, 1, H), w2, b2.reshape(E, 1, D))


def _gather_body(yg_hbm, s1_hbm, s2_hbm, c1_hbm, c2_hbm, buf1, buf2, i1, i2, sem):
    wid = lax.axis_index("s") * NC + lax.axis_index("c")
    base = wid * RPW
    for p in range(RPW // HR):
        b = base + p * HR
        pltpu.sync_copy(s1_hbm.at[pl.ds(b, HR)], i1)
        pltpu.sync_copy(s2_hbm.at[pl.ds(b, HR)], i2)
        pltpu.async_copy(yg_hbm.at[i1], buf1, sem).wait()
        pltpu.async_copy(yg_hbm.at[i2], buf2, sem).wait()
        pltpu.sync_copy(buf1, c1_hbm.at[pl.ds(b, HR)])
        pltpu.sync_copy(buf2, c2_hbm.at[pl.ds(b, HR)])


def _gather_pairs(yg, s1, s2):
    mesh = plsc.VectorSubcoreMesh(core_axis_name="c", subcore_axis_name="s")
    fn = functools.partial(
        pl.kernel, mesh=mesh,
        out_type=[
            jax.ShapeDtypeStruct((T, D // 2), jnp.int32),
            jax.ShapeDtypeStruct((T, D // 2), jnp.int32),
        ],
        scratch_types=[
            pltpu.VMEM((HR, D // 2), jnp.int32),
            pltpu.VMEM((HR, D // 2), jnp.int32),
            pltpu.VMEM((HR,), jnp.int32),
            pltpu.VMEM((HR,), jnp.int32),
            pltpu.SemaphoreType.DMA,
        ],
    )(_gather_body)
    return fn(yg, s1, s2)


BM = 256


def _mix_body(c1_ref, c2_ref, g1_ref, g2_ref, out_ref):
    c1 = _unpack(c1_ref[...])
    c2 = _unpack(c2_ref[...])
    out_ref[...] = (g1_ref[...] * c1.astype(jnp.float32) +
                    g2_ref[...] * c2.astype(jnp.float32))


def _mix(c1, c2, g1, g2):
    return pl.pallas_call(
        _mix_body,
        grid=(T // BM,),
        in_specs=[
            pl.BlockSpec((BM, D // 2), lambda i: (i, 0)),
            pl.BlockSpec((BM, D // 2), lambda i: (i, 0)),
            pl.BlockSpec((BM, 1), lambda i: (i, 0)),
            pl.BlockSpec((BM, 1), lambda i: (i, 0)),
        ],
        out_specs=pl.BlockSpec((BM, D), lambda i: (i, 0)),
        out_shape=jax.ShapeDtypeStruct((T, D), jnp.float32),
    )(c1, c2, g1, g2)


def kernel(x, gate_w, gate_b, bias, w1, b1, w2, b2):
    Bsz, S, Dx = x.shape
    flat = x.reshape(-1, Dx)
    s1, s2, g1, g2, be, xh = _route(flat, gate_w, gate_b, bias)
    s1 = s1.reshape(T)
    s2 = s2.reshape(T)
    xg = _dispatch(xh, s1, s2)
    yg = _grouped_ffn(be.reshape(NB), xg, w1, b1, w2, b2)
    c1, c2 = _gather_pairs(yg, s1, s2)
    out = _mix(c1, c2, g1, g2)
    return out.reshape(Bsz, S, Dx)


# speedup vs baseline: 1.0574x; 1.0574x over previous
import functools

import jax
import jax.numpy as jnp
from jax import lax
from jax.experimental import pallas as pl
from jax.experimental.pallas import tpu as pltpu
from jax.experimental.pallas import tpu_sc as plsc

E = 8
K = 2
T = 2048
D = 1024
H = 2048
BC = 256
NB = T * K // BC + E
P = NB * BC
NC = 2
NS = 16
NW = NC * NS
RPW = T // NW
HR = RPW // 2
CDT = jnp.bfloat16


def _silu(h):
    return h * (1.0 / (1.0 + jnp.exp(-h)))


def _pack(xbf):
    n, d = xbf.shape
    lo = lax.bitcast_convert_type(xbf[:, :d // 2],
                                  jnp.int16).astype(jnp.int32) & 0xFFFF
    hi = lax.bitcast_convert_type(xbf[:, d // 2:],
                                  jnp.int16).astype(jnp.int32) << 16
    return lo | hi


def _unpack(xi):
    lo = lax.bitcast_convert_type((xi & 0xFFFF).astype(jnp.int16),
                                  jnp.bfloat16)
    hi = lax.bitcast_convert_type(
        lax.shift_right_logical(xi, 16).astype(jnp.int16), jnp.bfloat16)
    return jnp.concatenate([lo, hi], axis=1)


def _route_body(x_ref, gw_ref, gb_ref, bias_ref,
                s1_ref, s2_ref, g1_ref, g2_ref, be_ref, xh_ref):
    x = x_ref[...]
    xbf = x.astype(jnp.bfloat16)
    xh_ref[...] = _pack(xbf)
    scores = lax.dot_general(
        xbf, gw_ref[...].astype(jnp.bfloat16),
        (((1,), (1,)), ((), ())), preferred_element_type=jnp.float32)
    scores = scores + gb_ref[...] + bias_ref[...]
    lane = lax.broadcasted_iota(jnp.int32, (T, E), 1)
    m1 = jnp.max(scores, axis=1, keepdims=True)
    i1 = jnp.min(jnp.where(scores == m1, lane, E), axis=1, keepdims=True)
    sel1 = lane == i1
    scores2 = jnp.where(sel1, -jnp.inf, scores)
    m2 = jnp.max(scores2, axis=1, keepdims=True)
    i2 = jnp.min(jnp.where(scores2 == m2, lane, E), axis=1, keepdims=True)
    sel2 = lane == i2
    tt = jnp.exp(m2 - m1)
    g1_ref[...] = 1.0 / (1.0 + tt)
    g2_ref[...] = tt / (1.0 + tt)

    mask = jnp.where(sel1 | sel2, 1.0, 0.0)
    TB = 512
    r_i = lax.broadcasted_iota(jnp.int32, (TB, TB), 0)
    c_i = lax.broadcasted_iota(jnp.int32, (TB, TB), 1)
    ltri = jnp.where(c_i < r_i, 1.0, 0.0).astype(jnp.float32)
    carry = jnp.zeros((1, E), jnp.float32)
    ranks = []
    for j in range(T // TB):
        mb = mask[j * TB:(j + 1) * TB, :]
        rb = lax.dot_general(ltri, mb, (((1,), (0,)), ((), ())),
                             preferred_element_type=jnp.float32) + carry
        ranks.append(rb)
        carry = carry + jnp.sum(mb, axis=0, keepdims=True)
    rank = jnp.concatenate(ranks, axis=0)
    counts = carry
    padded = jnp.floor((counts + (BC - 1)) * (1.0 / BC)) * BC
    e_i = lax.broadcasted_iota(jnp.int32, (E, E), 0)
    e_j = lax.broadcasted_iota(jnp.int32, (E, E), 1)
    utri = jnp.where(e_i < e_j, 1.0, 0.0).astype(jnp.float32)
    off = lax.dot_general(padded, utri, (((1,), (0,)), ((), ())),
                          preferred_element_type=jnp.float32)
    bnd = (off + padded) * (1.0 / BC)
    b_i = lax.broadcasted_iota(jnp.int32, (NB, E), 0).astype(jnp.float32)
    be = jnp.sum(jnp.where(b_i >= bnd, 1.0, 0.0), axis=1, keepdims=True)
    be_ref[...] = jnp.minimum(be, E - 1).astype(jnp.int32)

    rank_o = rank + off
    s1 = jnp.sum(jnp.where(sel1, rank_o, 0.0), axis=1, keepdims=True)
    s2 = jnp.sum(jnp.where(sel2, rank_o, 0.0), axis=1, keepdims=True)
    s1_ref[...] = s1.astype(jnp.int32)
    s2_ref[...] = s2.astype(jnp.int32)


def _route(flat, gate_w, gate_b, bias):
    return pl.pallas_call(
        _route_body,
        out_shape=[
            jax.ShapeDtypeStruct((T, 1), jnp.int32),
            jax.ShapeDtypeStruct((T, 1), jnp.int32),
            jax.ShapeDtypeStruct((T, 1), jnp.float32),
            jax.ShapeDtypeStruct((T, 1), jnp.float32),
            jax.ShapeDtypeStruct((NB, 1), jnp.int32),
            jax.ShapeDtypeStruct((T, D // 2), jnp.int32),
        ],
    )(flat, gate_w, gate_b.reshape(1, E), bias.reshape(1, E))


def _dispatch_body(x_hbm, s1_hbm, s2_hbm, xg_hbm, xbuf, i1, i2, sem):
    wid = lax.axis_index("s") * NC + lax.axis_index("c")
    base = wid * RPW
    pltpu.sync_copy(x_hbm.at[pl.ds(base, RPW)], xbuf)
    pltpu.sync_copy(s1_hbm.at[pl.ds(base, RPW)], i1)
    pltpu.sync_copy(s2_hbm.at[pl.ds(base, RPW)], i2)
    pltpu.async_copy(xbuf, xg_hbm.at[i1], sem).wait()
    pltpu.async_copy(xbuf, xg_hbm.at[i2], sem).wait()


def _dispatch(flat, s1, s2):
    mesh = plsc.VectorSubcoreMesh(core_axis_name="c", subcore_axis_name="s")
    fn = functools.partial(
        pl.kernel, mesh=mesh,
        out_type=jax.ShapeDtypeStruct((P, D // 2), jnp.int32),
        scratch_types=[
            pltpu.VMEM((RPW, D // 2), jnp.int32),
            pltpu.VMEM((RPW,), jnp.int32),
            pltpu.VMEM((RPW,), jnp.int32),
            pltpu.SemaphoreType.DMA,
        ],
    )(_dispatch_body)
    return fn(flat, s1, s2)


def _ffn_body(be_ref, xg_ref, w1_ref, b1_ref, w2_ref, b2_ref, yg_ref):
    xb = _unpack(xg_ref[...])
    h = lax.dot_general(xb, w1_ref[0].astype(CDT),
                        (((1,), (0,)), ((), ())),
                        preferred_element_type=jnp.float32)
    h = _silu(h + b1_ref[0])
    y = lax.dot_general(h.astype(CDT), w2_ref[0].astype(CDT),
                        (((1,), (0,)), ((), ())),
                        preferred_element_type=jnp.float32)
    yb = (y + b2_ref[0]).astype(CDT)
    yg_ref[...] = _pack(yb)


def _grouped_ffn(be, xg, w1, b1, w2, b2):
    grid_spec = pltpu.PrefetchScalarGridSpec(
        num_scalar_prefetch=1,
        grid=(NB,),
        in_specs=[
            pl.BlockSpec((BC, D // 2), lambda b, be: (b, 0)),
            pl.BlockSpec((1, D, H), lambda b, be: (be[b], 0, 0)),
            pl.BlockSpec((1, 1, H), lambda b, be: (be[b], 0, 0)),
            pl.BlockSpec((1, H, D), lambda b, be: (be[b], 0, 0)),
            pl.BlockSpec((1, 1, D), lambda b, be: (be[b], 0, 0)),
        ],
        out_specs=pl.BlockSpec((BC, D // 2), lambda b, be: (b, 0)),
    )
    return pl.pallas_call(
        _ffn_body,
        grid_spec=grid_spec,
        out_shape=jax.ShapeDtypeStruct((P, D // 2), jnp.int32),
    )(be, xg, w1, b1.reshape(E, 1, H), w2, b2.reshape(E, 1, D))


def _gather_body(yg_hbm, s1_hbm, s2_hbm, c1_hbm, c2_hbm, buf1, buf2, i1, i2, sem):
    wid = lax.axis_index("s") * NC + lax.axis_index("c")
    base = wid * RPW
    for p in range(RPW // HR):
        b = base + p * HR
        pltpu.sync_copy(s1_hbm.at[pl.ds(b, HR)], i1)
        pltpu.sync_copy(s2_hbm.at[pl.ds(b, HR)], i2)
        pltpu.async_copy(yg_hbm.at[i1], buf1, sem).wait()
        pltpu.async_copy(yg_hbm.at[i2], buf2, sem).wait()
        pltpu.sync_copy(buf1, c1_hbm.at[pl.ds(b, HR)])
        pltpu.sync_copy(buf2, c2_hbm.at[pl.ds(b, HR)])


def _gather_pairs(yg, s1, s2):
    mesh = plsc.VectorSubcoreMesh(core_axis_name="c", subcore_axis_name="s")
    fn = functools.partial(
        pl.kernel, mesh=mesh,
        out_type=[
            jax.ShapeDtypeStruct((T, D // 2), jnp.int32),
            jax.ShapeDtypeStruct((T, D // 2), jnp.int32),
        ],
        scratch_types=[
            pltpu.VMEM((HR, D // 2), jnp.int32),
            pltpu.VMEM((HR, D // 2), jnp.int32),
            pltpu.VMEM((HR,), jnp.int32),
            pltpu.VMEM((HR,), jnp.int32),
            pltpu.SemaphoreType.DMA,
        ],
    )(_gather_body)
    return fn(yg, s1, s2)


def _mix_body(c1_ref, c2_ref, g1_ref, g2_ref, out_ref):
    c1 = _unpack(c1_ref[...])
    c2 = _unpack(c2_ref[...])
    out_ref[...] = (g1_ref[...] * c1.astype(jnp.float32) +
                    g2_ref[...] * c2.astype(jnp.float32))


def _mix(c1, c2, g1, g2):
    return pl.pallas_call(
        _mix_body,
        out_shape=jax.ShapeDtypeStruct((T, D), jnp.float32),
    )(c1, c2, g1, g2)


def kernel(x, gate_w, gate_b, bias, w1, b1, w2, b2):
    Bsz, S, Dx = x.shape
    flat = x.reshape(-1, Dx)
    s1, s2, g1, g2, be, xh = _route(flat, gate_w, gate_b, bias)
    s1 = s1.reshape(T)
    s2 = s2.reshape(T)
    xg = _dispatch(xh, s1, s2)
    yg = _grouped_ffn(be.reshape(NB), xg, w1, b1, w2, b2)
    c1, c2 = _gather_pairs(yg, s1, s2)
    out = _mix(c1, c2, g1, g2)
    return out.reshape(Bsz, S, Dx)


# speedup vs baseline: 1.1209x; 1.0601x over previous
import functools

import jax
import jax.numpy as jnp
from jax import lax
from jax.experimental import pallas as pl
from jax.experimental.pallas import tpu as pltpu
from jax.experimental.pallas import tpu_sc as plsc

E = 8
K = 2
T = 2048
D = 1024
H = 2048
BC = 512
NB = T * K // BC + E
P = NB * BC
NC = 2
NS = 16
NW = NC * NS
RPW = T // NW
HR = RPW // 2
CDT = jnp.bfloat16


def _silu(h):
    return h * (1.0 / (1.0 + jnp.exp(-h)))


def _pack(xbf):
    n, d = xbf.shape
    lo = lax.bitcast_convert_type(xbf[:, :d // 2],
                                  jnp.int16).astype(jnp.int32) & 0xFFFF
    hi = lax.bitcast_convert_type(xbf[:, d // 2:],
                                  jnp.int16).astype(jnp.int32) << 16
    return lo | hi


def _unpack(xi):
    lo = lax.bitcast_convert_type((xi & 0xFFFF).astype(jnp.int16),
                                  jnp.bfloat16)
    hi = lax.bitcast_convert_type(
        lax.shift_right_logical(xi, 16).astype(jnp.int16), jnp.bfloat16)
    return jnp.concatenate([lo, hi], axis=1)


def _route_body(x_ref, gw_ref, gb_ref, bias_ref,
                s1_ref, s2_ref, g1_ref, g2_ref, be_ref, xh_ref):
    x = x_ref[...]
    xbf = x.astype(jnp.bfloat16)
    xh_ref[...] = _pack(xbf)
    scores = lax.dot_general(
        xbf, gw_ref[...].astype(jnp.bfloat16),
        (((1,), (1,)), ((), ())), preferred_element_type=jnp.float32)
    scores = scores + gb_ref[...] + bias_ref[...]
    lane = lax.broadcasted_iota(jnp.int32, (T, E), 1)
    m1 = jnp.max(scores, axis=1, keepdims=True)
    i1 = jnp.min(jnp.where(scores == m1, lane, E), axis=1, keepdims=True)
    sel1 = lane == i1
    scores2 = jnp.where(sel1, -jnp.inf, scores)
    m2 = jnp.max(scores2, axis=1, keepdims=True)
    i2 = jnp.min(jnp.where(scores2 == m2, lane, E), axis=1, keepdims=True)
    sel2 = lane == i2
    tt = jnp.exp(m2 - m1)
    g1_ref[...] = 1.0 / (1.0 + tt)
    g2_ref[...] = tt / (1.0 + tt)

    mask = jnp.where(sel1 | sel2, 1.0, 0.0)
    TB = 512
    r_i = lax.broadcasted_iota(jnp.int32, (TB, TB), 0)
    c_i = lax.broadcasted_iota(jnp.int32, (TB, TB), 1)
    ltri = jnp.where(c_i < r_i, 1.0, 0.0).astype(jnp.float32)
    carry = jnp.zeros((1, E), jnp.float32)
    ranks = []
    for j in range(T // TB):
        mb = mask[j * TB:(j + 1) * TB, :]
        rb = lax.dot_general(ltri, mb, (((1,), (0,)), ((), ())),
                             preferred_element_type=jnp.float32) + carry
        ranks.append(rb)
        carry = carry + jnp.sum(mb, axis=0, keepdims=True)
    rank = jnp.concatenate(ranks, axis=0)
    counts = carry
    padded = jnp.floor((counts + (BC - 1)) * (1.0 / BC)) * BC
    e_i = lax.broadcasted_iota(jnp.int32, (E, E), 0)
    e_j = lax.broadcasted_iota(jnp.int32, (E, E), 1)
    utri = jnp.where(e_i < e_j, 1.0, 0.0).astype(jnp.float32)
    off = lax.dot_general(padded, utri, (((1,), (0,)), ((), ())),
                          preferred_element_type=jnp.float32)
    bnd = (off + padded) * (1.0 / BC)
    b_i = lax.broadcasted_iota(jnp.int32, (NB, E), 0).astype(jnp.float32)
    be = jnp.sum(jnp.where(b_i >= bnd, 1.0, 0.0), axis=1, keepdims=True)
    be_ref[...] = jnp.minimum(be, E - 1).astype(jnp.int32)

    rank_o = rank + off
    s1 = jnp.sum(jnp.where(sel1, rank_o, 0.0), axis=1, keepdims=True)
    s2 = jnp.sum(jnp.where(sel2, rank_o, 0.0), axis=1, keepdims=True)
    s1_ref[...] = s1.astype(jnp.int32)
    s2_ref[...] = s2.astype(jnp.int32)


def _route(flat, gate_w, gate_b, bias):
    return pl.pallas_call(
        _route_body,
        out_shape=[
            jax.ShapeDtypeStruct((T, 1), jnp.int32),
            jax.ShapeDtypeStruct((T, 1), jnp.int32),
            jax.ShapeDtypeStruct((T, 1), jnp.float32),
            jax.ShapeDtypeStruct((T, 1), jnp.float32),
            jax.ShapeDtypeStruct((NB, 1), jnp.int32),
            jax.ShapeDtypeStruct((T, D // 2), jnp.int32),
        ],
    )(flat, gate_w, gate_b.reshape(1, E), bias.reshape(1, E))


def _dispatch_body(x_hbm, s1_hbm, s2_hbm, xg_hbm, xbuf, i1, i2, sem):
    wid = lax.axis_index("s") * NC + lax.axis_index("c")
    base = wid * RPW
    pltpu.sync_copy(x_hbm.at[pl.ds(base, RPW)], xbuf)
    pltpu.sync_copy(s1_hbm.at[pl.ds(base, RPW)], i1)
    pltpu.sync_copy(s2_hbm.at[pl.ds(base, RPW)], i2)
    pltpu.async_copy(xbuf, xg_hbm.at[i1], sem).wait()
    pltpu.async_copy(xbuf, xg_hbm.at[i2], sem).wait()


def _dispatch(flat, s1, s2):
    mesh = plsc.VectorSubcoreMesh(core_axis_name="c", subcore_axis_name="s")
    fn = functools.partial(
        pl.kernel, mesh=mesh,
        out_type=jax.ShapeDtypeStruct((P, D // 2), jnp.int32),
        scratch_types=[
            pltpu.VMEM((RPW, D // 2), jnp.int32),
            pltpu.VMEM((RPW,), jnp.int32),
            pltpu.VMEM((RPW,), jnp.int32),
            pltpu.SemaphoreType.DMA,
        ],
    )(_dispatch_body)
    return fn(flat, s1, s2)


def _ffn_body(be_ref, xg_ref, w1_ref, b1_ref, w2_ref, b2_ref, yg_ref):
    xb = _unpack(xg_ref[...])
    h = lax.dot_general(xb, w1_ref[0].astype(CDT),
                        (((1,), (0,)), ((), ())),
                        preferred_element_type=jnp.float32)
    h = _silu(h + b1_ref[0])
    y = lax.dot_general(h.astype(CDT), w2_ref[0].astype(CDT),
                        (((1,), (0,)), ((), ())),
                        preferred_element_type=jnp.float32)
    yb = (y + b2_ref[0]).astype(CDT)
    yg_ref[...] = _pack(yb)


def _grouped_ffn(be, xg, w1, b1, w2, b2):
    grid_spec = pltpu.PrefetchScalarGridSpec(
        num_scalar_prefetch=1,
        grid=(NB,),
        in_specs=[
            pl.BlockSpec((BC, D // 2), lambda b, be: (b, 0)),
            pl.BlockSpec((1, D, H), lambda b, be: (be[b], 0, 0)),
            pl.BlockSpec((1, 1, H), lambda b, be: (be[b], 0, 0)),
            pl.BlockSpec((1, H, D), lambda b, be: (be[b], 0, 0)),
            pl.BlockSpec((1, 1, D), lambda b, be: (be[b], 0, 0)),
        ],
        out_specs=pl.BlockSpec((BC, D // 2), lambda b, be: (b, 0)),
    )
    return pl.pallas_call(
        _ffn_body,
        grid_spec=grid_spec,
        out_shape=jax.ShapeDtypeStruct((P, D // 2), jnp.int32),
    )(be, xg, w1, b1.reshape(E, 1, H), w2, b2.reshape(E, 1, D))


def _gather_body(yg_hbm, s1_hbm, s2_hbm, c1_hbm, c2_hbm, buf1, buf2, i1, i2, sem):
    wid = lax.axis_index("s") * NC + lax.axis_index("c")
    base = wid * RPW
    for p in range(RPW // HR):
        b = base + p * HR
        pltpu.sync_copy(s1_hbm.at[pl.ds(b, HR)], i1)
        pltpu.sync_copy(s2_hbm.at[pl.ds(b, HR)], i2)
        pltpu.async_copy(yg_hbm.at[i1], buf1, sem).wait()
        pltpu.async_copy(yg_hbm.at[i2], buf2, sem).wait()
        pltpu.sync_copy(buf1, c1_hbm.at[pl.ds(b, HR)])
        pltpu.sync_copy(buf2, c2_hbm.at[pl.ds(b, HR)])


def _gather_pairs(yg, s1, s2):
    mesh = plsc.VectorSubcoreMesh(core_axis_name="c", subcore_axis_name="s")
    fn = functools.partial(
        pl.kernel, mesh=mesh,
        out_type=[
            jax.ShapeDtypeStruct((T, D // 2), jnp.int32),
            jax.ShapeDtypeStruct((T, D // 2), jnp.int32),
        ],
        scratch_types=[
            pltpu.VMEM((HR, D // 2), jnp.int32),
            pltpu.VMEM((HR, D // 2), jnp.int32),
            pltpu.VMEM((HR,), jnp.int32),
            pltpu.VMEM((HR,), jnp.int32),
            pltpu.SemaphoreType.DMA,
        ],
    )(_gather_body)
    return fn(yg, s1, s2)


def _mix_body(c1_ref, c2_ref, g1_ref, g2_ref, out_ref):
    c1 = _unpack(c1_ref[...])
    c2 = _unpack(c2_ref[...])
    out_ref[...] = (g1_ref[...] * c1.astype(jnp.float32) +
                    g2_ref[...] * c2.astype(jnp.float32))


def _mix(c1, c2, g1, g2):
    return pl.pallas_call(
        _mix_body,
        out_shape=jax.ShapeDtypeStruct((T, D), jnp.float32),
    )(c1, c2, g1, g2)


def kernel(x, gate_w, gate_b, bias, w1, b1, w2, b2):
    Bsz, S, Dx = x.shape
    flat = x.reshape(-1, Dx)
    s1, s2, g1, g2, be, xh = _route(flat, gate_w, gate_b, bias)
    s1 = s1.reshape(T)
    s2 = s2.reshape(T)
    xg = _dispatch(xh, s1, s2)
    yg = _grouped_ffn(be.reshape(NB), xg, w1, b1, w2, b2)
    c1, c2 = _gather_pairs(yg, s1, s2)
    out = _mix(c1, c2, g1, g2)
    return out.reshape(Bsz, S, Dx)
